# Initial kernel scaffold; baseline (speedup 1.0000x reference)
import jax
import jax.numpy as jnp
from jax.experimental import pallas as pl


def kernel(z, W):
    raise NotImplementedError("write your pallas kernel here")


# speedup vs baseline: 1.9817x; 1.9817x over previous
import functools

import jax
import jax.numpy as jnp
from jax.experimental import pallas as pl

_B = 16
_C = 64
_HW = 1024
_K = 1024
_BETA = 0.25


def _vq_body(z_ref, w_ref, zq_ref, idx_ref, loss_ref):
    b = pl.program_id(0)
    zb = z_ref[0]
    w = w_ref[...]
    s = jax.lax.dot_general(w, zb, (((1,), (0,)), ((), ())),
                            preferred_element_type=jnp.float32)
    w2 = jnp.sum(w * w, axis=1, keepdims=True)
    z2 = jnp.sum(zb * zb, axis=0, keepdims=True)
    d = (z2 + w2) - 2.0 * s
    m = jnp.min(d, axis=0, keepdims=True)
    ii = jax.lax.broadcasted_iota(jnp.int32, (_K, _HW), 0)
    idx = jnp.min(jnp.where(d == m, ii, _K), axis=0).astype(jnp.int32)
    idx_ref[0, 0, :] = idx
    part = jnp.sum(m, axis=1, keepdims=True)

    @pl.when(b == 0)
    def _():
        loss_ref[...] = jnp.zeros((1, 1), jnp.float32)

    loss_ref[...] += part
    oh = (ii == idx[None, :]).astype(jnp.float32)
    zq = jax.lax.dot_general(w, oh, (((0,), (0,)), ((), ())),
                             preferred_element_type=jnp.float32)
    zq_ref[0] = zb + (zq - zb)


@functools.partial(jax.jit, static_argnames=("interpret",))
def _vq_tc(z3, W, interpret=False):
    return pl.pallas_call(
        _vq_body,
        grid=(_B,),
        in_specs=[
            pl.BlockSpec((1, _C, _HW), lambda b: (b, 0, 0)),
            pl.BlockSpec((_K, _C), lambda b: (0, 0)),
        ],
        out_specs=[
            pl.BlockSpec((1, _C, _HW), lambda b: (b, 0, 0)),
            pl.BlockSpec((1, 1, _HW), lambda b: (b, 0, 0)),
            pl.BlockSpec((1, 1), lambda b: (0, 0)),
        ],
        out_shape=[
            jax.ShapeDtypeStruct((_B, _C, _HW), jnp.float32),
            jax.ShapeDtypeStruct((_B, 1, _HW), jnp.int32),
            jax.ShapeDtypeStruct((1, 1), jnp.float32),
        ],
        interpret=interpret,
    )(z3, W)


def kernel(z, W):
    z3 = z.reshape(_B, _C, _HW)
    zq3, idx3, loss = _vq_tc(z3, W)
    vq_loss = loss[0, 0] * ((1.0 + _BETA) / (_B * _C * _HW))
    return zq3.reshape(z.shape), vq_loss, idx3.reshape(_B * _HW)
